# Initial kernel scaffold; baseline (speedup 1.0000x reference)
import jax
import jax.numpy as jnp
from jax.experimental import pallas as pl


def kernel(src_node_features, neighbor_node_features, W_agg, b):
    raise NotImplementedError("write your pallas kernel here")


# speedup vs baseline: 1.3171x; 1.3171x over previous
import jax
import jax.numpy as jnp
from jax.experimental import pallas as pl

DEG = 32
D = 128
BLK = 400


def _body(src_ref, neigh_ref, w_ref, b_ref, out_ref):
    aggr = jnp.sum(neigh_ref[...], axis=1) * (1.0 / DEG)
    h = jnp.dot(aggr, w_ref[...], preferred_element_type=jnp.float32)
    h = h + jnp.dot(src_ref[...], b_ref[...], preferred_element_type=jnp.float32)
    out_ref[...] = jnp.maximum(h, 0.0)


def kernel(src_node_features, neighbor_node_features, W_agg, b):
    n = src_node_features.shape[0]
    grid = (n // BLK,)
    return pl.pallas_call(
        _body,
        grid=grid,
        in_specs=[
            pl.BlockSpec((BLK, D), lambda i: (i, 0)),
            pl.BlockSpec((BLK, DEG, D), lambda i: (i, 0, 0)),
            pl.BlockSpec((D, D), lambda i: (0, 0)),
            pl.BlockSpec((D, D), lambda i: (0, 0)),
        ],
        out_specs=pl.BlockSpec((BLK, D), lambda i: (i, 0)),
        out_shape=jax.ShapeDtypeStruct((n, D), jnp.float32),
    )(src_node_features, neighbor_node_features, W_agg, b)
